# Initial kernel scaffold; baseline (speedup 1.0000x reference)
import jax
import jax.numpy as jnp
from jax.experimental import pallas as pl


def kernel(node_features, edge_index_seq, sc_w1, sc_b1, sc_w2, sc_b2, w_ih_l0, w_hh_l0, b_ih_l0, b_hh_l0, w_ih_l1, w_hh_l1, b_ih_l1, b_hh_l1, in_proj_w, in_proj_b, out_w, out_b, p_w1, p_b1, p_w2, p_b2, p_w3, p_b3):
    raise NotImplementedError("write your pallas kernel here")


# speedup vs baseline: 26.0250x; 26.0250x over previous
import jax
import jax.numpy as jnp
from jax.experimental import pallas as pl
from jax.experimental.pallas import tpu as pltpu

B, T, N, F, H, C, E = 2, 8, 1024, 128, 256, 13, 16384
BH = B * H
EC = 2048
NTILE = 128
NT = N // NTILE
ROWS = B * NTILE

f32 = jnp.float32
bf16 = jnp.bfloat16


def _mm(a, w_bf):
    return jax.lax.dot_general(
        a.astype(bf16), w_bf, (((1,), (0,)), ((), ())),
        preferred_element_type=f32)


def _k1_body(x_ref, w1t_ref, b1_ref, w2t_ref, b2_ref, out_ref):
    x = x_ref[0, 0]
    h1 = jnp.maximum(_mm(x, w1t_ref[...]) + b1_ref[...], 0.0)
    out_ref[0] = _mm(h1, w2t_ref[...]) + b2_ref[...]


def _node_mlp(node_features, w1t, b1, w2t, b2):
    return pl.pallas_call(
        _k1_body,
        grid=(T, B),
        in_specs=[
            pl.BlockSpec((1, 1, N, F), lambda t, b: (b, t, 0, 0)),
            pl.BlockSpec((F, H), lambda t, b: (0, 0)),
            pl.BlockSpec((1, H), lambda t, b: (0, 0)),
            pl.BlockSpec((H, H), lambda t, b: (0, 0)),
            pl.BlockSpec((1, H), lambda t, b: (0, 0)),
        ],
        out_specs=pl.BlockSpec((1, N, H), lambda t, b: (t, 0, b)),
        out_shape=jax.ShapeDtypeStruct((T, N, BH), f32),
        compiler_params=pltpu.CompilerParams(
            dimension_semantics=("parallel", "parallel")),
    )(node_features, w1t, b1, w2t, b2)


def _k2_body(h_ref, src_ref, dst_ref, out_ref):
    h = h_ref[0]
    hb = h.astype(bf16)
    riota = jax.lax.broadcasted_iota(jnp.int32, (N, EC), 0)
    adj = jnp.zeros((N, N), f32)
    for c in range(E // EC):
        sc_ = src_ref[0, c]
        dc = dst_ref[0, c]
        ost = (sc_[None, :] == riota).astype(bf16)
        odt = (dc[None, :] == riota).astype(bf16)
        adj = adj + jax.lax.dot_general(
            odt, ost, (((1,), (1,)), ((), ())), preferred_element_type=f32)
    cnt = jnp.sum(adj, axis=1, keepdims=True)
    seg = jax.lax.dot_general(
        adj.astype(bf16), hb, (((1,), (0,)), ((), ())),
        preferred_element_type=f32)
    mean = seg / jnp.maximum(cnt, 1.0)
    out_ref[0] = jnp.where(cnt > 0.0, (h + mean) * 0.5, h)


def _message_pass(h_all, src3, dst3):
    return pl.pallas_call(
        _k2_body,
        grid=(T,),
        in_specs=[
            pl.BlockSpec((1, N, BH), lambda t: (t, 0, 0)),
            pl.BlockSpec((1, E // EC, EC), lambda t: (t, 0, 0)),
            pl.BlockSpec((1, E // EC, EC), lambda t: (t, 0, 0)),
        ],
        out_specs=pl.BlockSpec((1, N, BH), lambda t: (t, 0, 0)),
        out_shape=jax.ShapeDtypeStruct((T, N, BH), f32),
        compiler_params=pltpu.CompilerParams(
            dimension_semantics=("parallel",)),
    )(h_all, src3, dst3)


def _k3_body(hn_ref, wih0_ref, whh0_ref, bl0_ref, wih1_ref, whh1_ref, bl1_ref,
             inpt_ref, inpb_ref, outt_ref, outb_ref,
             p1t_ref, p1b_ref, p2t_ref, p2b_ref, p3t_ref, p3b_ref, out_ref):
    xs = []
    for t in range(T):
        ht = hn_ref[t]
        xs.append(jnp.concatenate([ht[:, :H], ht[:, H:]], axis=0))

    def lstm(xs_in, wih, whh, bl):
        h = jnp.zeros((ROWS, H), f32)
        c = jnp.zeros((ROWS, H), f32)
        ys = []
        for t in range(T):
            g = _mm(xs_in[t], wih) + _mm(h, whh) + bl
            i = jax.nn.sigmoid(g[:, 0:H])
            f = jax.nn.sigmoid(g[:, H:2 * H])
            gg = jnp.tanh(g[:, 2 * H:3 * H])
            o = jax.nn.sigmoid(g[:, 3 * H:4 * H])
            c = f * c + i * gg
            h = o * jnp.tanh(c)
            ys.append(h)
        return ys

    l1 = lstm(xs, wih0_ref[...], whh0_ref[...], bl0_ref[...])
    l2 = lstm(l1, wih1_ref[...], whh1_ref[...], bl1_ref[...])

    inpt = inpt_ref[...]
    inpb = inpb_ref[...]
    nH = 4
    dh = H // nH
    q7 = _mm(l2[T - 1], inpt[:, 0:H]) + inpb[:, 0:H]
    ks = [_mm(l2[j], inpt[:, H:2 * H]) + inpb[:, H:2 * H] for j in range(T)]
    vs = [_mm(l2[j], inpt[:, 2 * H:3 * H]) + inpb[:, 2 * H:3 * H]
          for j in range(T)]

    g_mat = (jax.lax.broadcasted_iota(jnp.int32, (H, nH), 0) // dh ==
             jax.lax.broadcasted_iota(jnp.int32, (H, nH), 1)).astype(f32)
    gt_mat = (jax.lax.broadcasted_iota(jnp.int32, (nH, H), 0) ==
              jax.lax.broadcasted_iota(jnp.int32, (nH, H), 1) // dh).astype(f32)

    scale = 1.0 / (dh ** 0.5)
    ss = []
    for j in range(T):
        sj = jax.lax.dot_general(
            q7 * ks[j], g_mat, (((1,), (0,)), ((), ())),
            preferred_element_type=f32) * scale
        ss.append(sj)
    m = ss[0]
    for j in range(1, T):
        m = jnp.maximum(m, ss[j])
    es = [jnp.exp(sj - m) for sj in ss]
    den = es[0]
    for j in range(1, T):
        den = den + es[j]
    ctx = jnp.zeros((ROWS, H), f32)
    for j in range(T):
        wj = es[j] / den
        wb = jax.lax.dot_general(
            wj, gt_mat, (((1,), (0,)), ((), ())),
            preferred_element_type=f32)
        ctx = ctx + wb * vs[j]

    attn7 = _mm(ctx, outt_ref[...]) + outb_ref[...]
    h1 = jnp.maximum(_mm(attn7, p1t_ref[...]) + p1b_ref[...], 0.0)
    h2 = jnp.maximum(_mm(h1, p2t_ref[...]) + p2b_ref[...], 0.0)
    logits = _mm(h2, p3t_ref[...]) + p3b_ref[...]
    out_ref[0, 0] = logits[0:NTILE]
    out_ref[1, 0] = logits[NTILE:ROWS]


def _seq_head(h_new, wih0, whh0, bl0, wih1, whh1, bl1,
              inpt, inpb, outt, outb, p1t, p1b, p2t, p2b, p3t, p3b):
    full = lambda shape: pl.BlockSpec(shape, lambda i: tuple(0 for _ in shape))
    return pl.pallas_call(
        _k3_body,
        grid=(NT,),
        in_specs=[
            pl.BlockSpec((T, NTILE, BH), lambda i: (0, i, 0)),
            full((H, 4 * H)), full((H, 4 * H)), full((1, 4 * H)),
            full((H, 4 * H)), full((H, 4 * H)), full((1, 4 * H)),
            full((H, 3 * H)), full((1, 3 * H)),
            full((H, H)), full((1, H)),
            full((H, 2 * H)), full((1, 2 * H)),
            full((2 * H, H)), full((1, H)),
            full((H, 128)), full((1, 128)),
        ],
        out_specs=pl.BlockSpec((B, 1, NTILE, 128), lambda i: (0, i, 0, 0)),
        out_shape=jax.ShapeDtypeStruct((B, NT, NTILE, 128), f32),
        compiler_params=pltpu.CompilerParams(
            dimension_semantics=("parallel",)),
    )(h_new, wih0, whh0, bl0, wih1, whh1, bl1,
      inpt, inpb, outt, outb, p1t, p1b, p2t, p2b, p3t, p3b)


def kernel(node_features, edge_index_seq, sc_w1, sc_b1, sc_w2, sc_b2,
           w_ih_l0, w_hh_l0, b_ih_l0, b_hh_l0,
           w_ih_l1, w_hh_l1, b_ih_l1, b_hh_l1,
           in_proj_w, in_proj_b, out_w, out_b,
           p_w1, p_b1, p_w2, p_b2, p_w3, p_b3):
    w1t = sc_w1.T.astype(bf16)
    w2t = sc_w2.T.astype(bf16)
    b1 = sc_b1.reshape(1, H)
    b2 = sc_b2.reshape(1, H)
    src3 = edge_index_seq[:, 0, :].reshape(T, E // EC, EC)
    dst3 = edge_index_seq[:, 1, :].reshape(T, E // EC, EC)

    wih0 = w_ih_l0.T.astype(bf16)
    whh0 = w_hh_l0.T.astype(bf16)
    bl0 = (b_ih_l0 + b_hh_l0).reshape(1, 4 * H)
    wih1 = w_ih_l1.T.astype(bf16)
    whh1 = w_hh_l1.T.astype(bf16)
    bl1 = (b_ih_l1 + b_hh_l1).reshape(1, 4 * H)
    inpt = in_proj_w.T.astype(bf16)
    inpb = in_proj_b.reshape(1, 3 * H)
    outt = out_w.T.astype(bf16)
    outb = out_b.reshape(1, H)
    p1t = p_w1.T.astype(bf16)
    p1b = p_b1.reshape(1, 2 * H)
    p2t = p_w2.T.astype(bf16)
    p2b = p_b2.reshape(1, H)
    p3t = jnp.zeros((H, 128), bf16).at[:, :C].set(p_w3.T.astype(bf16))
    p3b = jnp.zeros((1, 128), f32).at[0, :C].set(p_b3)

    h_all = _node_mlp(node_features, w1t, b1, w2t, b2)
    h_new = _message_pass(h_all, src3, dst3)
    out = _seq_head(h_new, wih0, whh0, bl0, wih1, whh1, bl1,
                    inpt, inpb, outt, outb, p1t, p1b, p2t, p2b, p3t, p3b)
    return out.reshape(B, N, 128)[:, :, :C]


# speedup vs baseline: 48.8687x; 1.8778x over previous
import functools

import jax
import jax.numpy as jnp
from jax import lax
from jax.experimental import pallas as pl
from jax.experimental.pallas import tpu as pltpu
from jax.experimental.pallas import tpu_sc as plsc

B, T, N, F, H, C, E = 2, 8, 1024, 128, 256, 13, 16384
BH = B * H
NTILE = 128
NT = N // NTILE
ROWS = B * NTILE

NSUB = 16
TPC = T // 2
EPS = E // NSUB
HALF = (N * N) // 2
SW2 = HALF // NSUB

f32 = jnp.float32
bf16 = jnp.bfloat16


def _mm(a, w_bf):
    return jax.lax.dot_general(
        a.astype(bf16), w_bf, (((1,), (0,)), ((), ())),
        preferred_element_type=f32)


def _adj_body(src_hbm, dst_hbm, out_hbm, srcv, dstv, idx_v, ones_v, zbuf,
              adj_sh):
    c = lax.axis_index("c")
    s = lax.axis_index("s")
    zv = jnp.zeros((16,), f32)
    ov = jnp.full((16,), 1.0, f32)
    for k in range(8):
        ones_v[pl.ds(k * 16, 16)] = ov

    @pl.loop(0, SW2, step=16)
    def _(i):
        zbuf[pl.ds(i, 16)] = zv

    dummy = HALF + s
    for tt in range(TPC):
        t = c * TPC + tt
        pltpu.sync_copy(src_hbm.at[t, pl.ds(s * EPS, EPS)], srcv)
        pltpu.sync_copy(dst_hbm.at[t, pl.ds(s * EPS, EPS)], dstv)
        for j in range(8):
            for k in range(8):
                w = j * 128 + k * 16
                d16 = dstv[pl.ds(w, 16)]
                s16 = srcv[pl.ds(w, 16)]
                flat = d16 * N + s16
                m0 = d16 < (N // 2)
                idx_v[j, pl.ds(k * 16, 16)] = jnp.where(m0, flat, dummy)
                idx_v[j + 8, pl.ds(k * 16, 16)] = jnp.where(
                    m0, dummy, flat - HALF)
        for half in range(2):
            pltpu.sync_copy(zbuf, adj_sh.at[pl.ds(s * SW2, SW2)])
            plsc.subcore_barrier()
            for j in range(8):
                pltpu.sync_copy(ones_v, adj_sh.at[idx_v.at[half * 8 + j]],
                                add=True)
            plsc.subcore_barrier()
            pltpu.sync_copy(adj_sh.at[pl.ds(s * SW2, SW2)],
                            out_hbm.at[t, pl.ds(half * HALF + s * SW2, SW2)])
            plsc.subcore_barrier()


def _adjacency(src2d, dst2d):
    mesh = plsc.VectorSubcoreMesh(core_axis_name="c", subcore_axis_name="s")
    k = functools.partial(
        pl.kernel, mesh=mesh,
        out_type=jax.ShapeDtypeStruct((T, N * N), f32),
        scratch_types=[
            pltpu.VMEM((EPS,), jnp.int32),
            pltpu.VMEM((EPS,), jnp.int32),
            pltpu.VMEM((16, 128), jnp.int32),
            pltpu.VMEM((128,), f32),
            pltpu.VMEM((SW2,), f32),
            pltpu.VMEM_SHARED((HALF + 16,), f32),
        ])(_adj_body)
    return k(src2d, dst2d)


def _k1_body(x_ref, w1t_ref, b1_ref, w2t_ref, b2_ref, out_ref):
    x = x_ref[0, 0]
    h1 = jnp.maximum(_mm(x, w1t_ref[...]) + b1_ref[...], 0.0)
    out_ref[0] = (_mm(h1, w2t_ref[...]) + b2_ref[...]).astype(bf16)


def _node_mlp(node_features, w1t, b1, w2t, b2):
    return pl.pallas_call(
        _k1_body,
        grid=(T, B),
        in_specs=[
            pl.BlockSpec((1, 1, N, F), lambda t, b: (b, t, 0, 0)),
            pl.BlockSpec((F, H), lambda t, b: (0, 0)),
            pl.BlockSpec((1, H), lambda t, b: (0, 0)),
            pl.BlockSpec((H, H), lambda t, b: (0, 0)),
            pl.BlockSpec((1, H), lambda t, b: (0, 0)),
        ],
        out_specs=pl.BlockSpec((1, N, H), lambda t, b: (t, 0, b)),
        out_shape=jax.ShapeDtypeStruct((T, N, BH), bf16),
        compiler_params=pltpu.CompilerParams(
            dimension_semantics=("parallel", "parallel")),
    )(node_features, w1t, b1, w2t, b2)


def _k3_body(hbf_ref, adj_ref, wih0_ref, whh0_ref, bl0_ref,
             wih1_ref, whh1_ref, bl1_ref,
             inpt_ref, inpb_ref, outt_ref, outb_ref,
             p1t_ref, p1b_ref, p2t_ref, p2b_ref, p3t_ref, p3b_ref, out_ref):
    i = pl.program_id(0)
    xs = []
    for t in range(T):
        adjt = adj_ref[t]
        seg = jax.lax.dot_general(
            adjt.astype(bf16), hbf_ref[t], (((1,), (0,)), ((), ())),
            preferred_element_type=f32)
        cnt = jnp.sum(adjt, axis=1, keepdims=True)
        hrow = hbf_ref[t, pl.ds(i * NTILE, NTILE)].astype(f32)
        mean = seg / jnp.maximum(cnt, 1.0)
        ht = jnp.where(cnt > 0.0, (hrow + mean) * 0.5, hrow)
        xs.append(jnp.concatenate([ht[:, :H], ht[:, H:]], axis=0))

    def lstm(xs_in, wih, whh, bl):
        h = jnp.zeros((ROWS, H), f32)
        c = jnp.zeros((ROWS, H), f32)
        ys = []
        for t in range(T):
            g = _mm(xs_in[t], wih) + _mm(h, whh) + bl
            ig = jax.nn.sigmoid(g[:, 0:H])
            fg = jax.nn.sigmoid(g[:, H:2 * H])
            gg = jnp.tanh(g[:, 2 * H:3 * H])
            og = jax.nn.sigmoid(g[:, 3 * H:4 * H])
            c = fg * c + ig * gg
            h = og * jnp.tanh(c)
            ys.append(h)
        return ys

    l1 = lstm(xs, wih0_ref[...], whh0_ref[...], bl0_ref[...])
    l2 = lstm(l1, wih1_ref[...], whh1_ref[...], bl1_ref[...])

    inpt = inpt_ref[...]
    inpb = inpb_ref[...]
    nH = 4
    dh = H // nH
    q7 = _mm(l2[T - 1], inpt[:, 0:H]) + inpb[:, 0:H]
    ks = [_mm(l2[j], inpt[:, H:2 * H]) + inpb[:, H:2 * H] for j in range(T)]
    vs = [_mm(l2[j], inpt[:, 2 * H:3 * H]) + inpb[:, 2 * H:3 * H]
          for j in range(T)]

    g_mat = (jax.lax.broadcasted_iota(jnp.int32, (H, nH), 0) // dh ==
             jax.lax.broadcasted_iota(jnp.int32, (H, nH), 1)).astype(f32)
    gt_mat = (jax.lax.broadcasted_iota(jnp.int32, (nH, H), 0) ==
              jax.lax.broadcasted_iota(jnp.int32, (nH, H), 1) // dh).astype(f32)

    scale = 1.0 / (dh ** 0.5)
    ss = []
    for j in range(T):
        sj = jax.lax.dot_general(
            q7 * ks[j], g_mat, (((1,), (0,)), ((), ())),
            preferred_element_type=f32) * scale
        ss.append(sj)
    m = ss[0]
    for j in range(1, T):
        m = jnp.maximum(m, ss[j])
    es = [jnp.exp(sj - m) for sj in ss]
    den = es[0]
    for j in range(1, T):
        den = den + es[j]
    ctx = jnp.zeros((ROWS, H), f32)
    for j in range(T):
        wj = es[j] / den
        wb = jax.lax.dot_general(
            wj, gt_mat, (((1,), (0,)), ((), ())),
            preferred_element_type=f32)
        ctx = ctx + wb * vs[j]

    attn7 = _mm(ctx, outt_ref[...]) + outb_ref[...]
    h1 = jnp.maximum(_mm(attn7, p1t_ref[...]) + p1b_ref[...], 0.0)
    h2 = jnp.maximum(_mm(h1, p2t_ref[...]) + p2b_ref[...], 0.0)
    logits = _mm(h2, p3t_ref[...]) + p3b_ref[...]
    out_ref[0, 0] = logits[0:NTILE]
    out_ref[1, 0] = logits[NTILE:ROWS]


def _seq_head(h_bf, adj, wih0, whh0, bl0, wih1, whh1, bl1,
              inpt, inpb, outt, outb, p1t, p1b, p2t, p2b, p3t, p3b):
    full = lambda shape: pl.BlockSpec(shape, lambda i: tuple(0 for _ in shape))
    return pl.pallas_call(
        _k3_body,
        grid=(NT,),
        in_specs=[
            full((T, N, BH)),
            pl.BlockSpec((T, NTILE, N), lambda i: (0, i, 0)),
            full((H, 4 * H)), full((H, 4 * H)), full((1, 4 * H)),
            full((H, 4 * H)), full((H, 4 * H)), full((1, 4 * H)),
            full((H, 3 * H)), full((1, 3 * H)),
            full((H, H)), full((1, H)),
            full((H, 2 * H)), full((1, 2 * H)),
            full((2 * H, H)), full((1, H)),
            full((H, 128)), full((1, 128)),
        ],
        out_specs=pl.BlockSpec((B, 1, NTILE, 128), lambda i: (0, i, 0, 0)),
        out_shape=jax.ShapeDtypeStruct((B, NT, NTILE, 128), f32),
        compiler_params=pltpu.CompilerParams(
            dimension_semantics=("arbitrary",)),
    )(h_bf, adj, wih0, whh0, bl0, wih1, whh1, bl1,
      inpt, inpb, outt, outb, p1t, p1b, p2t, p2b, p3t, p3b)


def kernel(node_features, edge_index_seq, sc_w1, sc_b1, sc_w2, sc_b2,
           w_ih_l0, w_hh_l0, b_ih_l0, b_hh_l0,
           w_ih_l1, w_hh_l1, b_ih_l1, b_hh_l1,
           in_proj_w, in_proj_b, out_w, out_b,
           p_w1, p_b1, p_w2, p_b2, p_w3, p_b3):
    w1t = sc_w1.T.astype(bf16)
    w2t = sc_w2.T.astype(bf16)
    b1 = sc_b1.reshape(1, H)
    b2 = sc_b2.reshape(1, H)
    src2d = edge_index_seq[:, 0, :]
    dst2d = edge_index_seq[:, 1, :]

    wih0 = w_ih_l0.T.astype(bf16)
    whh0 = w_hh_l0.T.astype(bf16)
    bl0 = (b_ih_l0 + b_hh_l0).reshape(1, 4 * H)
    wih1 = w_ih_l1.T.astype(bf16)
    whh1 = w_hh_l1.T.astype(bf16)
    bl1 = (b_ih_l1 + b_hh_l1).reshape(1, 4 * H)
    inpt = in_proj_w.T.astype(bf16)
    inpb = in_proj_b.reshape(1, 3 * H)
    outt = out_w.T.astype(bf16)
    outb = out_b.reshape(1, H)
    p1t = p_w1.T.astype(bf16)
    p1b = p_b1.reshape(1, 2 * H)
    p2t = p_w2.T.astype(bf16)
    p2b = p_b2.reshape(1, H)
    p3t = jnp.zeros((H, 128), bf16).at[:, :C].set(p_w3.T.astype(bf16))
    p3b = jnp.zeros((1, 128), f32).at[0, :C].set(p_b3)

    adj = _adjacency(src2d, dst2d).reshape(T, N, N)
    h_bf = _node_mlp(node_features, w1t, b1, w2t, b2)
    out = _seq_head(h_bf, adj, wih0, whh0, bl0, wih1, whh1, bl1,
                    inpt, inpb, outt, outb, p1t, p1b, p2t, p2b, p3t, p3b)
    return out.reshape(B, N, 128)[:, :, :C]


# speedup vs baseline: 59.7126x; 1.2219x over previous
import functools

import jax
import jax.numpy as jnp
from jax import lax
from jax.experimental import pallas as pl
from jax.experimental.pallas import tpu as pltpu
from jax.experimental.pallas import tpu_sc as plsc

B, T, N, F, H, C, E = 2, 8, 1024, 128, 256, 13, 16384
BH = B * H
NTILE = 128
NT = N // NTILE
ROWS = B * NTILE

NSUB = 16
TPC = T // 2
EPS = E // NSUB
HALF = (N * N) // 2
SW2 = HALF // NSUB

f32 = jnp.float32
bf16 = jnp.bfloat16


def _mm(a, w_bf):
    return jax.lax.dot_general(
        a.astype(bf16), w_bf, (((1,), (0,)), ((), ())),
        preferred_element_type=f32)


def _adj_body(edges_hbm, out_hbm, srcv, dstv, idx_v, ones_v, zeros_v, zbuf,
              adj_sh, sem):
    c = lax.axis_index("c")
    s = lax.axis_index("s")
    zv = jnp.zeros((16,), f32)
    ov = jnp.full((16,), 1.0, f32)
    for k in range(8):
        ones_v[pl.ds(k * 16, 16)] = ov
        zeros_v[pl.ds(k * 16, 16)] = zv

    @pl.loop(0, SW2, step=16)
    def _(i):
        zbuf[pl.ds(i, 16)] = zv

    pltpu.sync_copy(zbuf, adj_sh.at[pl.ds(s * SW2, SW2)])
    plsc.subcore_barrier()

    rps = SW2 // N
    dummy = HALF + s
    for tt in range(TPC):
        t = c * TPC + tt
        pltpu.sync_copy(edges_hbm.at[t, 0, pl.ds(s * EPS, EPS)], srcv)
        pltpu.sync_copy(edges_hbm.at[t, 1, pl.ds(s * EPS, EPS)], dstv)
        for j in range(8):
            for k in range(8):
                w = j * 128 + k * 16
                d16 = dstv[pl.ds(w, 16)]
                s16 = srcv[pl.ds(w, 16)]
                flat = d16 * N + s16
                m0 = d16 < (N // 2)
                idx_v[j, pl.ds(k * 16, 16)] = jnp.where(m0, flat, dummy)
                idx_v[j + 8, pl.ds(k * 16, 16)] = jnp.where(
                    m0, dummy, flat - HALF)
        for half in range(2):
            for j in range(8):
                pltpu.sync_copy(ones_v, adj_sh.at[idx_v.at[half * 8 + j]],
                                add=True)
            plsc.subcore_barrier()
            row0 = half * (N // 2) + s * rps
            handles = [
                pltpu.async_copy(adj_sh.at[pl.ds(s * SW2 + r * N, N)],
                                 out_hbm.at[t, row0 + r], sem)
                for r in range(rps)
            ]
            for hnd in handles:
                hnd.wait()
            plsc.subcore_barrier()
            for j in range(8):
                pltpu.sync_copy(zeros_v, adj_sh.at[idx_v.at[half * 8 + j]])
            plsc.subcore_barrier()


def _adjacency(edge_index_seq):
    mesh = plsc.VectorSubcoreMesh(core_axis_name="c", subcore_axis_name="s")
    k = functools.partial(
        pl.kernel, mesh=mesh,
        out_type=jax.ShapeDtypeStruct((T, N, N), f32),
        scratch_types=[
            pltpu.VMEM((EPS,), jnp.int32),
            pltpu.VMEM((EPS,), jnp.int32),
            pltpu.VMEM((16, 128), jnp.int32),
            pltpu.VMEM((128,), f32),
            pltpu.VMEM((128,), f32),
            pltpu.VMEM((SW2,), f32),
            pltpu.VMEM_SHARED((HALF + 16,), f32),
            pltpu.SemaphoreType.DMA,
        ])(_adj_body)
    return k(edge_index_seq)


def _k1_body(x_ref, w1t_ref, b1_ref, w2t_ref, b2_ref, out_ref):
    x = x_ref[0, 0]
    h1 = jnp.maximum(_mm(x, w1t_ref[...]) + b1_ref[...], 0.0)
    out_ref[0] = (_mm(h1, w2t_ref[...]) + b2_ref[...]).astype(bf16)


def _node_mlp(node_features, w1t, b1, w2t, b2):
    return pl.pallas_call(
        _k1_body,
        grid=(T, B),
        in_specs=[
            pl.BlockSpec((1, 1, N, F), lambda t, b: (b, t, 0, 0)),
            pl.BlockSpec((F, H), lambda t, b: (0, 0)),
            pl.BlockSpec((1, H), lambda t, b: (0, 0)),
            pl.BlockSpec((H, H), lambda t, b: (0, 0)),
            pl.BlockSpec((1, H), lambda t, b: (0, 0)),
        ],
        out_specs=pl.BlockSpec((1, N, H), lambda t, b: (t, 0, b)),
        out_shape=jax.ShapeDtypeStruct((T, N, BH), bf16),
        compiler_params=pltpu.CompilerParams(
            dimension_semantics=("parallel", "parallel")),
    )(node_features, w1t, b1, w2t, b2)


def _k3_body(hbf_ref, adj_ref, wih0_ref, whh0_ref, bl0_ref,
             wih1_ref, whh1_ref, bl1_ref,
             inpt_ref, inpb_ref, outt_ref, outb_ref,
             p1t_ref, p1b_ref, p2t_ref, p2b_ref, p3t_ref, p3b_ref, out_ref):
    i = pl.program_id(0)
    xs = []
    for t in range(T):
        adjt = adj_ref[t]
        seg = jax.lax.dot_general(
            adjt.astype(bf16), hbf_ref[t], (((1,), (0,)), ((), ())),
            preferred_element_type=f32)
        cnt = jnp.sum(adjt, axis=1, keepdims=True)
        hrow = hbf_ref[t, pl.ds(i * NTILE, NTILE)].astype(f32)
        mean = seg / jnp.maximum(cnt, 1.0)
        ht = jnp.where(cnt > 0.0, (hrow + mean) * 0.5, hrow)
        xs.append(jnp.concatenate([ht[:, :H], ht[:, H:]], axis=0))

    def lstm(xs_in, wih, whh, bl):
        h = jnp.zeros((ROWS, H), f32)
        c = jnp.zeros((ROWS, H), f32)
        ys = []
        for t in range(T):
            g = _mm(xs_in[t], wih) + _mm(h, whh) + bl
            ig = jax.nn.sigmoid(g[:, 0:H])
            fg = jax.nn.sigmoid(g[:, H:2 * H])
            gg = jnp.tanh(g[:, 2 * H:3 * H])
            og = jax.nn.sigmoid(g[:, 3 * H:4 * H])
            c = fg * c + ig * gg
            h = og * jnp.tanh(c)
            ys.append(h)
        return ys

    l1 = lstm(xs, wih0_ref[...], whh0_ref[...], bl0_ref[...])
    l2 = lstm(l1, wih1_ref[...], whh1_ref[...], bl1_ref[...])

    inpt = inpt_ref[...]
    inpb = inpb_ref[...]
    nH = 4
    dh = H // nH
    q7 = _mm(l2[T - 1], inpt[:, 0:H]) + inpb[:, 0:H]
    ks = [_mm(l2[j], inpt[:, H:2 * H]) + inpb[:, H:2 * H] for j in range(T)]
    vs = [_mm(l2[j], inpt[:, 2 * H:3 * H]) + inpb[:, 2 * H:3 * H]
          for j in range(T)]

    g_mat = (jax.lax.broadcasted_iota(jnp.int32, (H, nH), 0) // dh ==
             jax.lax.broadcasted_iota(jnp.int32, (H, nH), 1)).astype(f32)
    gt_mat = (jax.lax.broadcasted_iota(jnp.int32, (nH, H), 0) ==
              jax.lax.broadcasted_iota(jnp.int32, (nH, H), 1) // dh).astype(f32)

    scale = 1.0 / (dh ** 0.5)
    ss = []
    for j in range(T):
        sj = jax.lax.dot_general(
            q7 * ks[j], g_mat, (((1,), (0,)), ((), ())),
            preferred_element_type=f32) * scale
        ss.append(sj)
    m = ss[0]
    for j in range(1, T):
        m = jnp.maximum(m, ss[j])
    es = [jnp.exp(sj - m) for sj in ss]
    den = es[0]
    for j in range(1, T):
        den = den + es[j]
    ctx = jnp.zeros((ROWS, H), f32)
    for j in range(T):
        wj = es[j] / den
        wb = jax.lax.dot_general(
            wj, gt_mat, (((1,), (0,)), ((), ())),
            preferred_element_type=f32)
        ctx = ctx + wb * vs[j]

    attn7 = _mm(ctx, outt_ref[...]) + outb_ref[...]
    h1 = jnp.maximum(_mm(attn7, p1t_ref[...]) + p1b_ref[...], 0.0)
    h2 = jnp.maximum(_mm(h1, p2t_ref[...]) + p2b_ref[...], 0.0)
    logits = _mm(h2, p3t_ref[...]) + p3b_ref[...]
    out_ref[0, 0] = logits[0:NTILE]
    out_ref[1, 0] = logits[NTILE:ROWS]


def _seq_head(h_bf, adj, wih0, whh0, bl0, wih1, whh1, bl1,
              inpt, inpb, outt, outb, p1t, p1b, p2t, p2b, p3t, p3b):
    full = lambda shape: pl.BlockSpec(shape, lambda i: tuple(0 for _ in shape))
    return pl.pallas_call(
        _k3_body,
        grid=(NT,),
        in_specs=[
            full((T, N, BH)),
            pl.BlockSpec((T, NTILE, N), lambda i: (0, i, 0)),
            full((H, 4 * H)), full((H, 4 * H)), full((1, 4 * H)),
            full((H, 4 * H)), full((H, 4 * H)), full((1, 4 * H)),
            full((H, 3 * H)), full((1, 3 * H)),
            full((H, H)), full((1, H)),
            full((H, 2 * H)), full((1, 2 * H)),
            full((2 * H, H)), full((1, H)),
            full((H, 128)), full((1, 128)),
        ],
        out_specs=pl.BlockSpec((B, 1, NTILE, 128), lambda i: (0, i, 0, 0)),
        out_shape=jax.ShapeDtypeStruct((B, NT, NTILE, 128), f32),
        compiler_params=pltpu.CompilerParams(
            dimension_semantics=("parallel",)),
    )(h_bf, adj, wih0, whh0, bl0, wih1, whh1, bl1,
      inpt, inpb, outt, outb, p1t, p1b, p2t, p2b, p3t, p3b)


def kernel(node_features, edge_index_seq, sc_w1, sc_b1, sc_w2, sc_b2,
           w_ih_l0, w_hh_l0, b_ih_l0, b_hh_l0,
           w_ih_l1, w_hh_l1, b_ih_l1, b_hh_l1,
           in_proj_w, in_proj_b, out_w, out_b,
           p_w1, p_b1, p_w2, p_b2, p_w3, p_b3):
    w1t = sc_w1.T.astype(bf16)
    w2t = sc_w2.T.astype(bf16)
    b1 = sc_b1.reshape(1, H)
    b2 = sc_b2.reshape(1, H)

    wih0 = w_ih_l0.T.astype(bf16)
    whh0 = w_hh_l0.T.astype(bf16)
    bl0 = (b_ih_l0 + b_hh_l0).reshape(1, 4 * H)
    wih1 = w_ih_l1.T.astype(bf16)
    whh1 = w_hh_l1.T.astype(bf16)
    bl1 = (b_ih_l1 + b_hh_l1).reshape(1, 4 * H)
    inpt = in_proj_w.T.astype(bf16)
    inpb = in_proj_b.reshape(1, 3 * H)
    outt = out_w.T.astype(bf16)
    outb = out_b.reshape(1, H)
    p1t = p_w1.T.astype(bf16)
    p1b = p_b1.reshape(1, 2 * H)
    p2t = p_w2.T.astype(bf16)
    p2b = p_b2.reshape(1, H)
    p3t = jnp.zeros((H, 128), bf16).at[:, :C].set(p_w3.T.astype(bf16))
    p3b = jnp.zeros((1, 128), f32).at[0, :C].set(p_b3)

    adj = _adjacency(edge_index_seq)
    h_bf = _node_mlp(node_features, w1t, b1, w2t, b2)
    out = _seq_head(h_bf, adj, wih0, whh0, bl0, wih1, whh1, bl1,
                    inpt, inpb, outt, outb, p1t, p1b, p2t, p2b, p3t, p3b)
    return out.reshape(B, N, 128)[:, :, :C]


# speedup vs baseline: 60.3560x; 1.0108x over previous
import functools

import jax
import jax.numpy as jnp
from jax import lax
from jax.experimental import pallas as pl
from jax.experimental.pallas import tpu as pltpu
from jax.experimental.pallas import tpu_sc as plsc

B, T, N, F, H, C, E = 2, 8, 1024, 128, 256, 13, 16384
BH = B * H
NTILE = 128
NT = N // NTILE
ROWS = B * NTILE

NSUB = 16
TPC = T // 2
EPS = E // NSUB
HALF = (N * N) // 2
SW2 = HALF // NSUB

f32 = jnp.float32
bf16 = jnp.bfloat16


def _mm(a, w_bf):
    return jax.lax.dot_general(
        a.astype(bf16), w_bf, (((1,), (0,)), ((), ())),
        preferred_element_type=f32)


def _adj_body(edges_hbm, out_hbm, srcv, dstv, idx_v, ones_v, zbuf,
              adj_sh, sem):
    c = lax.axis_index("c")
    s = lax.axis_index("s")
    zv = jnp.zeros((16,), f32)
    ov = jnp.full((16,), 1.0, f32)
    for k in range(8):
        ones_v[pl.ds(k * 16, 16)] = ov

    @pl.loop(0, SW2, step=16)
    def _(i):
        zbuf[pl.ds(i, 16)] = zv

    pltpu.sync_copy(zbuf, adj_sh.at[pl.ds(s * SW2, SW2)])
    plsc.subcore_barrier()

    rps = SW2 // N
    dummy = HALF + s
    for tt in range(TPC):
        t = c * TPC + tt
        pltpu.sync_copy(edges_hbm.at[t, 0, pl.ds(s * EPS, EPS)], srcv)
        pltpu.sync_copy(edges_hbm.at[t, 1, pl.ds(s * EPS, EPS)], dstv)
        for j in range(8):
            for k in range(8):
                w = j * 128 + k * 16
                d16 = dstv[pl.ds(w, 16)]
                s16 = srcv[pl.ds(w, 16)]
                flat = d16 * N + s16
                m0 = d16 < (N // 2)
                idx_v[j, pl.ds(k * 16, 16)] = jnp.where(m0, flat, dummy)
                idx_v[j + 8, pl.ds(k * 16, 16)] = jnp.where(
                    m0, dummy, flat - HALF)
        for half in range(2):
            adds = [
                pltpu.async_copy(ones_v, adj_sh.at[idx_v.at[half * 8 + j]],
                                 sem, add=True)
                for j in range(8)
            ]
            for hnd in adds:
                hnd.wait()
            plsc.subcore_barrier()
            row0 = half * (N // 2) + s * rps
            handles = [
                pltpu.async_copy(adj_sh.at[pl.ds(s * SW2 + r * N, N)],
                                 out_hbm.at[t, row0 + r], sem)
                for r in range(rps)
            ]
            for hnd in handles:
                hnd.wait()
            pltpu.sync_copy(zbuf, adj_sh.at[pl.ds(s * SW2, SW2)])
            plsc.subcore_barrier()


def _adjacency(edge_index_seq):
    mesh = plsc.VectorSubcoreMesh(core_axis_name="c", subcore_axis_name="s")
    k = functools.partial(
        pl.kernel, mesh=mesh,
        out_type=jax.ShapeDtypeStruct((T, N, N), f32),
        scratch_types=[
            pltpu.VMEM((EPS,), jnp.int32),
            pltpu.VMEM((EPS,), jnp.int32),
            pltpu.VMEM((16, 128), jnp.int32),
            pltpu.VMEM((128,), f32),
            pltpu.VMEM((SW2,), f32),
            pltpu.VMEM_SHARED((HALF + 16,), f32),
            pltpu.SemaphoreType.DMA,
        ])(_adj_body)
    return k(edge_index_seq)


def _k1_body(x_ref, w1t_ref, b1_ref, w2t_ref, b2_ref, out_ref):
    x = x_ref[0, 0]
    h1 = jnp.maximum(_mm(x, w1t_ref[...]) + b1_ref[...], 0.0)
    out_ref[0] = (_mm(h1, w2t_ref[...]) + b2_ref[...]).astype(bf16)


def _node_mlp(node_features, w1t, b1, w2t, b2):
    return pl.pallas_call(
        _k1_body,
        grid=(T, B),
        in_specs=[
            pl.BlockSpec((1, 1, N, F), lambda t, b: (b, t, 0, 0)),
            pl.BlockSpec((F, H), lambda t, b: (0, 0)),
            pl.BlockSpec((1, H), lambda t, b: (0, 0)),
            pl.BlockSpec((H, H), lambda t, b: (0, 0)),
            pl.BlockSpec((1, H), lambda t, b: (0, 0)),
        ],
        out_specs=pl.BlockSpec((1, N, H), lambda t, b: (t, 0, b)),
        out_shape=jax.ShapeDtypeStruct((T, N, BH), bf16),
        compiler_params=pltpu.CompilerParams(
            dimension_semantics=("parallel", "parallel")),
    )(node_features, w1t, b1, w2t, b2)


def _k3_body(hbf_ref, adj_ref, wih0_ref, whh0_ref, bl0_ref,
             wih1_ref, whh1_ref, bl1_ref,
             inpt_ref, inpb_ref, outt_ref, outb_ref,
             p1t_ref, p1b_ref, p2t_ref, p2b_ref, p3t_ref, p3b_ref, out_ref):
    i = pl.program_id(0)
    xs = []
    for t in range(T):
        adjt = adj_ref[t]
        seg = jax.lax.dot_general(
            adjt.astype(bf16), hbf_ref[t], (((1,), (0,)), ((), ())),
            preferred_element_type=f32)
        cnt = jnp.sum(adjt, axis=1, keepdims=True)
        hrow = hbf_ref[t, pl.ds(i * NTILE, NTILE)].astype(f32)
        mean = seg / jnp.maximum(cnt, 1.0)
        ht = jnp.where(cnt > 0.0, (hrow + mean) * 0.5, hrow)
        xs.append(jnp.concatenate([ht[:, :H], ht[:, H:]], axis=0))

    def lstm(xs_in, wih, whh, bl):
        h = jnp.zeros((ROWS, H), f32)
        c = jnp.zeros((ROWS, H), f32)
        ys = []
        for t in range(T):
            g = _mm(xs_in[t], wih) + _mm(h, whh) + bl
            ig = jax.nn.sigmoid(g[:, 0:H])
            fg = jax.nn.sigmoid(g[:, H:2 * H])
            gg = jnp.tanh(g[:, 2 * H:3 * H])
            og = jax.nn.sigmoid(g[:, 3 * H:4 * H])
            c = fg * c + ig * gg
            h = og * jnp.tanh(c)
            ys.append(h)
        return ys

    l1 = lstm(xs, wih0_ref[...], whh0_ref[...], bl0_ref[...])
    l2 = lstm(l1, wih1_ref[...], whh1_ref[...], bl1_ref[...])

    inpt = inpt_ref[...]
    inpb = inpb_ref[...]
    nH = 4
    dh = H // nH
    q7 = _mm(l2[T - 1], inpt[:, 0:H]) + inpb[:, 0:H]
    ks = [_mm(l2[j], inpt[:, H:2 * H]) + inpb[:, H:2 * H] for j in range(T)]
    vs = [_mm(l2[j], inpt[:, 2 * H:3 * H]) + inpb[:, 2 * H:3 * H]
          for j in range(T)]

    g_mat = (jax.lax.broadcasted_iota(jnp.int32, (H, nH), 0) // dh ==
             jax.lax.broadcasted_iota(jnp.int32, (H, nH), 1)).astype(f32)
    gt_mat = (jax.lax.broadcasted_iota(jnp.int32, (nH, H), 0) ==
              jax.lax.broadcasted_iota(jnp.int32, (nH, H), 1) // dh).astype(f32)

    scale = 1.0 / (dh ** 0.5)
    ss = []
    for j in range(T):
        sj = jax.lax.dot_general(
            q7 * ks[j], g_mat, (((1,), (0,)), ((), ())),
            preferred_element_type=f32) * scale
        ss.append(sj)
    m = ss[0]
    for j in range(1, T):
        m = jnp.maximum(m, ss[j])
    es = [jnp.exp(sj - m) for sj in ss]
    den = es[0]
    for j in range(1, T):
        den = den + es[j]
    ctx = jnp.zeros((ROWS, H), f32)
    for j in range(T):
        wj = es[j] / den
        wb = jax.lax.dot_general(
            wj, gt_mat, (((1,), (0,)), ((), ())),
            preferred_element_type=f32)
        ctx = ctx + wb * vs[j]

    attn7 = _mm(ctx, outt_ref[...]) + outb_ref[...]
    h1 = jnp.maximum(_mm(attn7, p1t_ref[...]) + p1b_ref[...], 0.0)
    h2 = jnp.maximum(_mm(h1, p2t_ref[...]) + p2b_ref[...], 0.0)
    logits = _mm(h2, p3t_ref[...]) + p3b_ref[...]
    out_ref[0, 0] = logits[0:NTILE]
    out_ref[1, 0] = logits[NTILE:ROWS]


def _seq_head(h_bf, adj, wih0, whh0, bl0, wih1, whh1, bl1,
              inpt, inpb, outt, outb, p1t, p1b, p2t, p2b, p3t, p3b):
    full = lambda shape: pl.BlockSpec(shape, lambda i: tuple(0 for _ in shape))
    return pl.pallas_call(
        _k3_body,
        grid=(NT,),
        in_specs=[
            full((T, N, BH)),
            pl.BlockSpec((T, NTILE, N), lambda i: (0, i, 0)),
            full((H, 4 * H)), full((H, 4 * H)), full((1, 4 * H)),
            full((H, 4 * H)), full((H, 4 * H)), full((1, 4 * H)),
            full((H, 3 * H)), full((1, 3 * H)),
            full((H, H)), full((1, H)),
            full((H, 2 * H)), full((1, 2 * H)),
            full((2 * H, H)), full((1, H)),
            full((H, 128)), full((1, 128)),
        ],
        out_specs=pl.BlockSpec((B, 1, NTILE, 128), lambda i: (0, i, 0, 0)),
        out_shape=jax.ShapeDtypeStruct((B, NT, NTILE, 128), f32),
        compiler_params=pltpu.CompilerParams(
            dimension_semantics=("parallel",)),
    )(h_bf, adj, wih0, whh0, bl0, wih1, whh1, bl1,
      inpt, inpb, outt, outb, p1t, p1b, p2t, p2b, p3t, p3b)


def kernel(node_features, edge_index_seq, sc_w1, sc_b1, sc_w2, sc_b2,
           w_ih_l0, w_hh_l0, b_ih_l0, b_hh_l0,
           w_ih_l1, w_hh_l1, b_ih_l1, b_hh_l1,
           in_proj_w, in_proj_b, out_w, out_b,
           p_w1, p_b1, p_w2, p_b2, p_w3, p_b3):
    w1t = sc_w1.T.astype(bf16)
    w2t = sc_w2.T.astype(bf16)
    b1 = sc_b1.reshape(1, H)
    b2 = sc_b2.reshape(1, H)

    wih0 = w_ih_l0.T.astype(bf16)
    whh0 = w_hh_l0.T.astype(bf16)
    bl0 = (b_ih_l0 + b_hh_l0).reshape(1, 4 * H)
    wih1 = w_ih_l1.T.astype(bf16)
    whh1 = w_hh_l1.T.astype(bf16)
    bl1 = (b_ih_l1 + b_hh_l1).reshape(1, 4 * H)
    inpt = in_proj_w.T.astype(bf16)
    inpb = in_proj_b.reshape(1, 3 * H)
    outt = out_w.T.astype(bf16)
    outb = out_b.reshape(1, H)
    p1t = p_w1.T.astype(bf16)
    p1b = p_b1.reshape(1, 2 * H)
    p2t = p_w2.T.astype(bf16)
    p2b = p_b2.reshape(1, H)
    p3t = jnp.zeros((H, 128), bf16).at[:, :C].set(p_w3.T.astype(bf16))
    p3b = jnp.zeros((1, 128), f32).at[0, :C].set(p_b3)

    adj = _adjacency(edge_index_seq)
    h_bf = _node_mlp(node_features, w1t, b1, w2t, b2)
    out = _seq_head(h_bf, adj, wih0, whh0, bl0, wih1, whh1, bl1,
                    inpt, inpb, outt, outb, p1t, p1b, p2t, p2b, p3t, p3b)
    return out.reshape(B, N, 128)[:, :, :C]


# speedup vs baseline: 62.6564x; 1.0381x over previous
import functools

import jax
import jax.numpy as jnp
from jax import lax
from jax.experimental import pallas as pl
from jax.experimental.pallas import tpu as pltpu
from jax.experimental.pallas import tpu_sc as plsc

B, T, N, F, H, C, E = 2, 8, 1024, 128, 256, 13, 16384
BH = B * H
NTILE = 128
NT = N // NTILE
ROWS = B * NTILE

NSUB = 16
TPC = T // 2
EPS = E // NSUB
HALF = (N * N) // 2
SW2 = HALF // NSUB

f32 = jnp.float32
bf16 = jnp.bfloat16


def _mm(a, w_bf):
    return jax.lax.dot_general(
        a.astype(bf16), w_bf, (((1,), (0,)), ((), ())),
        preferred_element_type=f32)


def _adj_body(edges_hbm, out_hbm, srcv, dstv, idx_v, ones_v, zbuf,
              adj_sh, sem):
    c = lax.axis_index("c")
    s = lax.axis_index("s")
    zv = jnp.zeros((16,), f32)
    ov = jnp.full((16,), 1.0, f32)
    for k in range(8):
        ones_v[pl.ds(k * 16, 16)] = ov

    @pl.loop(0, SW2, step=16)
    def _(i):
        zbuf[pl.ds(i, 16)] = zv

    pltpu.sync_copy(zbuf, adj_sh.at[pl.ds(s * SW2, SW2)])
    plsc.subcore_barrier()

    rps = SW2 // N
    dummy = HALF + s
    for tt in range(TPC):
        t = c * TPC + tt
        pltpu.sync_copy(edges_hbm.at[t, 0, pl.ds(s * EPS, EPS)], srcv)
        pltpu.sync_copy(edges_hbm.at[t, 1, pl.ds(s * EPS, EPS)], dstv)
        for j in range(8):
            for k in range(8):
                w = j * 128 + k * 16
                d16 = dstv[pl.ds(w, 16)]
                s16 = srcv[pl.ds(w, 16)]
                flat = d16 * N + s16
                m0 = d16 < (N // 2)
                idx_v[j, pl.ds(k * 16, 16)] = jnp.where(m0, flat, dummy)
                idx_v[j + 8, pl.ds(k * 16, 16)] = jnp.where(
                    m0, dummy, flat - HALF)
        for half in range(2):
            adds = [
                pltpu.async_copy(ones_v, adj_sh.at[idx_v.at[half * 8 + j]],
                                 sem, add=True)
                for j in range(8)
            ]
            for hnd in adds:
                hnd.wait()
            plsc.subcore_barrier()
            row0 = half * (N // 2) + s * rps
            handles = [
                pltpu.async_copy(adj_sh.at[pl.ds(s * SW2 + r * N, N)],
                                 out_hbm.at[t, row0 + r], sem)
                for r in range(rps)
            ]
            for hnd in handles:
                hnd.wait()
            pltpu.sync_copy(zbuf, adj_sh.at[pl.ds(s * SW2, SW2)])
            plsc.subcore_barrier()


def _adjacency(edge_index_seq):
    mesh = plsc.VectorSubcoreMesh(core_axis_name="c", subcore_axis_name="s")
    k = functools.partial(
        pl.kernel, mesh=mesh,
        out_type=jax.ShapeDtypeStruct((T, N, N), f32),
        scratch_types=[
            pltpu.VMEM((EPS,), jnp.int32),
            pltpu.VMEM((EPS,), jnp.int32),
            pltpu.VMEM((16, 128), jnp.int32),
            pltpu.VMEM((128,), f32),
            pltpu.VMEM((SW2,), f32),
            pltpu.VMEM_SHARED((HALF + 16,), f32),
            pltpu.SemaphoreType.DMA,
        ])(_adj_body)
    return k(edge_index_seq)


DH = BH + 16


def _k1_body(x_ref, w1t_ref, b1_ref, w2t_ref, b2_ref, out_ref):
    hs = []
    for b in range(B):
        x = x_ref[b, 0]
        h1 = jnp.maximum(_mm(x, w1t_ref[...]) + b1_ref[...], 0.0)
        hs.append((_mm(h1, w2t_ref[...]) + b2_ref[...]).astype(bf16))
    ones16 = jnp.full((N, 16), 1.0, bf16)
    out_ref[0] = jnp.concatenate(hs + [ones16], axis=1)


def _node_mlp(node_features, w1t, b1, w2t, b2):
    return pl.pallas_call(
        _k1_body,
        grid=(T,),
        in_specs=[
            pl.BlockSpec((B, 1, N, F), lambda t: (0, t, 0, 0)),
            pl.BlockSpec((F, H), lambda t: (0, 0)),
            pl.BlockSpec((1, H), lambda t: (0, 0)),
            pl.BlockSpec((H, H), lambda t: (0, 0)),
            pl.BlockSpec((1, H), lambda t: (0, 0)),
        ],
        out_specs=pl.BlockSpec((1, N, DH), lambda t: (t, 0, 0)),
        out_shape=jax.ShapeDtypeStruct((T, N, DH), bf16),
        compiler_params=pltpu.CompilerParams(
            dimension_semantics=("parallel",)),
    )(node_features, w1t, b1, w2t, b2)


def _k3_body(hbf_ref, adj_ref, wih0_ref, whh0_ref, bl0_ref,
             wih1_ref, whh1_ref, bl1_ref,
             inpt_ref, inpb_ref, outt_ref, outb_ref,
             p1t_ref, p1b_ref, p2t_ref, p2b_ref, p3t_ref, p3b_ref, out_ref):
    i = pl.program_id(0)

    def sig(x):
        return 0.5 * (jnp.tanh(0.5 * x) + 1.0)

    xs = []
    for t in range(T):
        adjt = adj_ref[t]
        seg = jax.lax.dot_general(
            adjt.astype(bf16), hbf_ref[t], (((1,), (0,)), ((), ())),
            preferred_element_type=f32)
        cnt = seg[:, BH:BH + 1]
        hrow = hbf_ref[t, pl.ds(i * NTILE, NTILE)].astype(f32)
        mean = seg[:, :BH] / jnp.maximum(cnt, 1.0)
        ht = jnp.where(cnt > 0.0, (hrow[:, :BH] + mean) * 0.5, hrow[:, :BH])
        xs.append(jnp.concatenate([ht[:, :H], ht[:, H:]], axis=0))

    def lstm(stack_in, wih, whh, bl):
        xp = _mm(stack_in, wih)
        h = jnp.zeros((ROWS, H), f32)
        c = jnp.zeros((ROWS, H), f32)
        ys = []
        for t in range(T):
            g = xp[t * ROWS:(t + 1) * ROWS] + _mm(h, whh) + bl
            ig = sig(g[:, 0:H])
            fg = sig(g[:, H:2 * H])
            gg = jnp.tanh(g[:, 2 * H:3 * H])
            og = sig(g[:, 3 * H:4 * H])
            c = fg * c + ig * gg
            h = og * jnp.tanh(c)
            ys.append(h)
        return ys

    l1 = lstm(jnp.concatenate(xs, axis=0), wih0_ref[...], whh0_ref[...],
              bl0_ref[...])
    l2 = lstm(jnp.concatenate(l1, axis=0), wih1_ref[...], whh1_ref[...],
              bl1_ref[...])

    inpt = inpt_ref[...]
    inpb = inpb_ref[...]
    nH = 4
    dh = H // nH
    l2s = jnp.concatenate(l2, axis=0)
    q7 = _mm(l2[T - 1], inpt[:, 0:H]) + inpb[:, 0:H]
    kvs = _mm(l2s, inpt[:, H:3 * H])
    ks = [kvs[j * ROWS:(j + 1) * ROWS, 0:H] + inpb[:, H:2 * H]
          for j in range(T)]
    vs = [kvs[j * ROWS:(j + 1) * ROWS, H:2 * H] + inpb[:, 2 * H:3 * H]
          for j in range(T)]

    g_mat = (jax.lax.broadcasted_iota(jnp.int32, (H, nH), 0) // dh ==
             jax.lax.broadcasted_iota(jnp.int32, (H, nH), 1)).astype(f32)
    gt_mat = (jax.lax.broadcasted_iota(jnp.int32, (nH, H), 0) ==
              jax.lax.broadcasted_iota(jnp.int32, (nH, H), 1) // dh).astype(f32)

    scale = 1.0 / (dh ** 0.5)
    ss = []
    for j in range(T):
        sj = jax.lax.dot_general(
            q7 * ks[j], g_mat, (((1,), (0,)), ((), ())),
            preferred_element_type=f32) * scale
        ss.append(sj)
    m = ss[0]
    for j in range(1, T):
        m = jnp.maximum(m, ss[j])
    es = [jnp.exp(sj - m) for sj in ss]
    den = es[0]
    for j in range(1, T):
        den = den + es[j]
    ctx = jnp.zeros((ROWS, H), f32)
    for j in range(T):
        wj = es[j] / den
        wb = jax.lax.dot_general(
            wj, gt_mat, (((1,), (0,)), ((), ())),
            preferred_element_type=f32)
        ctx = ctx + wb * vs[j]

    attn7 = _mm(ctx, outt_ref[...]) + outb_ref[...]
    h1 = jnp.maximum(_mm(attn7, p1t_ref[...]) + p1b_ref[...], 0.0)
    h2 = jnp.maximum(_mm(h1, p2t_ref[...]) + p2b_ref[...], 0.0)
    logits = _mm(h2, p3t_ref[...]) + p3b_ref[...]
    out_ref[0, 0] = logits[0:NTILE]
    out_ref[1, 0] = logits[NTILE:ROWS]


def _seq_head(h_bf, adj, wih0, whh0, bl0, wih1, whh1, bl1,
              inpt, inpb, outt, outb, p1t, p1b, p2t, p2b, p3t, p3b):
    full = lambda shape: pl.BlockSpec(shape, lambda i: tuple(0 for _ in shape))
    return pl.pallas_call(
        _k3_body,
        grid=(NT,),
        in_specs=[
            full((T, N, DH)),
            pl.BlockSpec((T, NTILE, N), lambda i: (0, i, 0)),
            full((H, 4 * H)), full((H, 4 * H)), full((1, 4 * H)),
            full((H, 4 * H)), full((H, 4 * H)), full((1, 4 * H)),
            full((H, 3 * H)), full((1, 3 * H)),
            full((H, H)), full((1, H)),
            full((H, 2 * H)), full((1, 2 * H)),
            full((2 * H, H)), full((1, H)),
            full((H, 128)), full((1, 128)),
        ],
        out_specs=pl.BlockSpec((B, 1, NTILE, 128), lambda i: (0, i, 0, 0)),
        out_shape=jax.ShapeDtypeStruct((B, NT, NTILE, 128), f32),
        compiler_params=pltpu.CompilerParams(
            dimension_semantics=("parallel",)),
    )(h_bf, adj, wih0, whh0, bl0, wih1, whh1, bl1,
      inpt, inpb, outt, outb, p1t, p1b, p2t, p2b, p3t, p3b)


def kernel(node_features, edge_index_seq, sc_w1, sc_b1, sc_w2, sc_b2,
           w_ih_l0, w_hh_l0, b_ih_l0, b_hh_l0,
           w_ih_l1, w_hh_l1, b_ih_l1, b_hh_l1,
           in_proj_w, in_proj_b, out_w, out_b,
           p_w1, p_b1, p_w2, p_b2, p_w3, p_b3):
    w1t = sc_w1.T.astype(bf16)
    w2t = sc_w2.T.astype(bf16)
    b1 = sc_b1.reshape(1, H)
    b2 = sc_b2.reshape(1, H)

    wih0 = w_ih_l0.T.astype(bf16)
    whh0 = w_hh_l0.T.astype(bf16)
    bl0 = (b_ih_l0 + b_hh_l0).reshape(1, 4 * H)
    wih1 = w_ih_l1.T.astype(bf16)
    whh1 = w_hh_l1.T.astype(bf16)
    bl1 = (b_ih_l1 + b_hh_l1).reshape(1, 4 * H)
    inpt = in_proj_w.T.astype(bf16)
    inpb = in_proj_b.reshape(1, 3 * H)
    outt = out_w.T.astype(bf16)
    outb = out_b.reshape(1, H)
    p1t = p_w1.T.astype(bf16)
    p1b = p_b1.reshape(1, 2 * H)
    p2t = p_w2.T.astype(bf16)
    p2b = p_b2.reshape(1, H)
    p3t = jnp.zeros((H, 128), bf16).at[:, :C].set(p_w3.T.astype(bf16))
    p3b = jnp.zeros((1, 128), f32).at[0, :C].set(p_b3)

    adj = _adjacency(edge_index_seq)
    h_bf = _node_mlp(node_features, w1t, b1, w2t, b2)
    out = _seq_head(h_bf, adj, wih0, whh0, bl0, wih1, whh1, bl1,
                    inpt, inpb, outt, outb, p1t, p1b, p2t, p2b, p3t, p3b)
    return out.reshape(B, N, 128)[:, :, :C]
